# baseline (speedup 1.0000x reference)
import jax
import jax.numpy as jnp
from jax import lax
from jax.experimental import pallas as pl
from jax.experimental.pallas import tpu as pltpu
from jax.experimental.pallas import tpu_sc as plsc


def _body(x_hbm, cls_hbm, x_v, cls_v):
    pltpu.sync_copy(x_hbm.at[0, 0], x_v.at[pl.ds(0, 3)])
    v = x_v[...]
    a = v[0]
    b = v[1]
    d = v[2]
    is0 = jnp.logical_and(a >= b, a >= d)
    is1 = jnp.logical_and(jnp.logical_not(is0), b >= d)
    idx = jnp.where(is0, jnp.int32(0), jnp.where(is1, jnp.int32(1), jnp.int32(2)))
    lane = lax.iota(jnp.int32, 16)
    out = jnp.where(lane == idx, jnp.float32(1.0), jnp.float32(0.1))
    cls_v[...] = out
    pltpu.sync_copy(cls_v.at[pl.ds(0, 3)], cls_hbm.at[0])


@jax.jit
def kernel(x):
    class_outputs = pl.kernel(
        _body,
        out_type=jax.ShapeDtypeStruct((1, 3), jnp.float32),
        mesh=plsc.VectorSubcoreMesh(
            core_axis_name="c",
            subcore_axis_name="s",
            num_cores=1,
            num_subcores=1,
        ),
        scratch_types=[
            pltpu.VMEM((16,), jnp.float32),
            pltpu.VMEM((16,), jnp.float32),
        ],
    )(x)
    encodings = jnp.ones((1, 1, 7), jnp.float32)
    return class_outputs, encodings


# speedup vs baseline: 1.0178x; 1.0178x over previous
import jax
import jax.numpy as jnp
from jax import lax
from jax.experimental import pallas as pl
from jax.experimental.pallas import tpu as pltpu
from jax.experimental.pallas import tpu_sc as plsc


def _body(x_hbm, cls_hbm, enc_hbm, x_v, cls_v, enc_v, sem_in, sem_enc):
    in_cp = pltpu.make_async_copy(x_hbm.at[0, 0], x_v.at[pl.ds(0, 3)], sem_in)
    in_cp.start()
    enc_v[...] = jnp.full((16,), 1.0, jnp.float32)
    enc_cp = pltpu.make_async_copy(enc_v.at[pl.ds(0, 7)], enc_hbm.at[0, 0], sem_enc)
    enc_cp.start()
    in_cp.wait()
    v = x_v[...]
    a = v[0]
    b = v[1]
    d = v[2]
    is0 = jnp.logical_and(a >= b, a >= d)
    is1 = jnp.logical_and(jnp.logical_not(is0), b >= d)
    idx = jnp.where(is0, jnp.int32(0), jnp.where(is1, jnp.int32(1), jnp.int32(2)))
    lane = lax.iota(jnp.int32, 16)
    out = jnp.where(lane == idx, jnp.float32(1.0), jnp.float32(0.1))
    cls_v[...] = out
    pltpu.sync_copy(cls_v.at[pl.ds(0, 3)], cls_hbm.at[0])
    enc_cp.wait()


@jax.jit
def kernel(x):
    return pl.kernel(
        _body,
        out_type=(
            jax.ShapeDtypeStruct((1, 3), jnp.float32),
            jax.ShapeDtypeStruct((1, 1, 7), jnp.float32),
        ),
        mesh=plsc.VectorSubcoreMesh(
            core_axis_name="c",
            subcore_axis_name="s",
            num_cores=1,
            num_subcores=1,
        ),
        scratch_types=[
            pltpu.VMEM((16,), jnp.float32),
            pltpu.VMEM((16,), jnp.float32),
            pltpu.VMEM((16,), jnp.float32),
            pltpu.SemaphoreType.DMA,
            pltpu.SemaphoreType.DMA,
        ],
    )(x)


# speedup vs baseline: 1.0188x; 1.0010x over previous
import jax
import jax.numpy as jnp
from jax import lax
from jax.experimental import pallas as pl
from jax.experimental.pallas import tpu as pltpu
from jax.experimental.pallas import tpu_sc as plsc


def _body(x_hbm, cls_hbm, enc_hbm, x_v, enc_v, sem_in, sem_enc):
    in_cp = pltpu.make_async_copy(x_hbm.at[0, 0], x_v.at[pl.ds(0, 3)], sem_in)
    in_cp.start()
    enc_v[...] = jnp.full((16,), 1.0, jnp.float32)
    enc_cp = pltpu.make_async_copy(enc_v.at[pl.ds(0, 7)], enc_hbm.at[0, 0], sem_enc)
    enc_cp.start()
    in_cp.wait()
    v = x_v[...]
    a = v[0]
    b = v[1]
    d = v[2]
    is0 = jnp.logical_and(a >= b, a >= d)
    is1 = jnp.logical_and(jnp.logical_not(is0), b >= d)
    idx = jnp.where(is0, jnp.int32(0), jnp.where(is1, jnp.int32(1), jnp.int32(2)))
    lane = lax.iota(jnp.int32, 16)
    out = jnp.where(lane == idx, jnp.float32(1.0), jnp.float32(0.1))
    x_v[...] = out
    pltpu.sync_copy(x_v.at[pl.ds(0, 3)], cls_hbm.at[0])
    enc_cp.wait()


@jax.jit
def kernel(x):
    return pl.kernel(
        _body,
        out_type=(
            jax.ShapeDtypeStruct((1, 3), jnp.float32),
            jax.ShapeDtypeStruct((1, 1, 7), jnp.float32),
        ),
        mesh=plsc.VectorSubcoreMesh(
            core_axis_name="c",
            subcore_axis_name="s",
            num_cores=1,
            num_subcores=1,
        ),
        scratch_types=[
            pltpu.VMEM((16,), jnp.float32),
            pltpu.VMEM((16,), jnp.float32),
            pltpu.SemaphoreType.DMA,
            pltpu.SemaphoreType.DMA,
        ],
    )(x)
